# Initial kernel scaffold; baseline (speedup 1.0000x reference)
import jax
import jax.numpy as jnp
from jax.experimental import pallas as pl


def kernel(x, edge_index, batch, Wr1, Wo1, b1, Wpr1, Wpo1, bp1, Wr2, Wo2, b2, Wpr2, Wpo2, bp2, Wr3, Wo3, b3, Wpr3, Wpo3, bp3, Wm, bm):
    raise NotImplementedError("write your pallas kernel here")


# speedup vs baseline: 33.5459x; 33.5459x over previous
import functools

import jax
import jax.numpy as jnp
from jax import lax
from jax.experimental import pallas as pl
from jax.experimental.pallas import tpu as pltpu
from jax.experimental.pallas import tpu_sc as plsc

N = 10000
E = 320000
F = 128
H = 256
NP = 10240
NC, NS = 2, 16
NW = NC * NS
RPW = 79
EP = NW * RPW * 128
ROWS_PER_TILE = NP // NS


def _k1_body(xpad, srcp, dstp, zeros2d, out, sidx, didx, rows, agg_sh, sem):
    cid = lax.axis_index("c")
    sid = lax.axis_index("s")
    w = cid * NS + sid
    pltpu.sync_copy(zeros2d.at[pl.ds(sid * ROWS_PER_TILE, ROWS_PER_TILE)],
                    agg_sh.at[pl.ds(sid * ROWS_PER_TILE, ROWS_PER_TILE)])
    pltpu.sync_copy(srcp.at[w], sidx)
    pltpu.sync_copy(dstp.at[w], didx)
    plsc.subcore_barrier()

    def body(j, carry):
        pltpu.async_copy(xpad.at[sidx.at[j]], rows, sem).wait()
        pltpu.sync_copy(rows, agg_sh.at[didx.at[j]], add=True)
        return carry

    lax.fori_loop(0, RPW, body, 0)
    plsc.subcore_barrier()

    def body2(t, carry):
        r0 = sid * ROWS_PER_TILE + t * 128
        pltpu.sync_copy(agg_sh.at[pl.ds(r0, 128)], rows)
        pltpu.sync_copy(rows, out.at[pl.ds(cid * NP + r0, 128)])
        return carry

    lax.fori_loop(0, ROWS_PER_TILE // 128, body2, 0)


_k1 = functools.partial(
    pl.kernel,
    out_type=jax.ShapeDtypeStruct((NC * NP, F), jnp.float32),
    mesh=plsc.VectorSubcoreMesh(core_axis_name="c", subcore_axis_name="s",
                                num_cores=NC, num_subcores=NS),
    scratch_types=[
        pltpu.VMEM((RPW, 128), jnp.int32),
        pltpu.VMEM((RPW, 128), jnp.int32),
        pltpu.VMEM((128, F), jnp.float32),
        pltpu.VMEM_SHARED((NP, F), jnp.float32),
        pltpu.SemaphoreType.DMA,
    ],
)(_k1_body)


def _k2_body(agg0, agg1, xb, wr, wo, b1r, wpr, wpo, bp1s, h_out, a_out, b_out):
    aggb = agg0[...] + agg1[...]
    h = jnp.dot(aggb, wr[...], preferred_element_type=jnp.float32)
    h += jnp.dot(xb[...], wo[...], preferred_element_type=jnp.float32)
    h = jnp.maximum(h + b1r[...], 0.0)
    h_out[...] = h
    a_out[...] = jnp.sum(h * wpr[...], axis=1).reshape(1, 1, -1)
    b_out[...] = (jnp.sum(h * wpo[...], axis=1) + bp1s[0, 0]).reshape(1, 1, -1)


def _k2(aggp, xpad, Wr1, Wo1, b1r, wpr1, wpo1, bp1s):
    R = 1024
    G = NP // R
    return pl.pallas_call(
        _k2_body,
        grid=(G,),
        in_specs=[
            pl.BlockSpec((R, F), lambda i: (i, 0)),
            pl.BlockSpec((R, F), lambda i: (i + G, 0)),
            pl.BlockSpec((R, F), lambda i: (i, 0)),
            pl.BlockSpec((F, H), lambda i: (0, 0)),
            pl.BlockSpec((F, H), lambda i: (0, 0)),
            pl.BlockSpec((1, H), lambda i: (0, 0)),
            pl.BlockSpec((1, H), lambda i: (0, 0)),
            pl.BlockSpec((1, H), lambda i: (0, 0)),
            pl.BlockSpec((1, 1), lambda i: (0, 0)),
        ],
        out_specs=[
            pl.BlockSpec((R, H), lambda i: (i, 0)),
            pl.BlockSpec((1, 1, R), lambda i: (i, 0, 0)),
            pl.BlockSpec((1, 1, R), lambda i: (i, 0, 0)),
        ],
        out_shape=[
            jax.ShapeDtypeStruct((NP, H), jnp.float32),
            jax.ShapeDtypeStruct((G, 1, R), jnp.float32),
            jax.ShapeDtypeStruct((G, 1, R), jnp.float32),
        ],
    )(aggp, aggp, xpad, Wr1, Wo1, b1r, wpr1, wpo1, bp1s)


def _k3_body(a_hbm, srcp, dstp, out, sidx, didx, vals, zb, score_sh, sem):
    cid = lax.axis_index("c")
    sid = lax.axis_index("s")
    w = cid * NS + sid

    @pl.when(sid == 0)
    def _zero():
        def zbody(i, carry):
            zb[pl.ds(i * 16, 16)] = jnp.zeros((16,), jnp.float32)
            return carry
        lax.fori_loop(0, NP // 16, zbody, 0)
        pltpu.sync_copy(zb, score_sh)

    pltpu.sync_copy(srcp.at[w], sidx)
    pltpu.sync_copy(dstp.at[w], didx)

    def gbody(j, carry):
        pltpu.async_copy(a_hbm.at[sidx.at[j]], vals.at[j], sem).wait()
        return carry

    lax.fori_loop(0, RPW, gbody, 0)
    plsc.subcore_barrier()

    def sbody(j, carry):
        pltpu.sync_copy(vals.at[j], score_sh.at[didx.at[j]], add=True)
        return carry

    lax.fori_loop(0, RPW, sbody, 0)
    plsc.subcore_barrier()

    @pl.when(sid == 0)
    def _out():
        pltpu.sync_copy(score_sh, zb)
        pltpu.sync_copy(zb, out.at[cid])


_k3 = functools.partial(
    pl.kernel,
    out_type=jax.ShapeDtypeStruct((NC, NP), jnp.float32),
    mesh=plsc.VectorSubcoreMesh(core_axis_name="c", subcore_axis_name="s",
                                num_cores=NC, num_subcores=NS),
    scratch_types=[
        pltpu.VMEM((RPW, 128), jnp.int32),
        pltpu.VMEM((RPW, 128), jnp.int32),
        pltpu.VMEM((RPW, 128), jnp.float32),
        pltpu.VMEM((NP,), jnp.float32),
        pltpu.VMEM_SHARED((NP,), jnp.float32),
        pltpu.SemaphoreType.DMA,
    ],
)(_k3_body)


def _k4_body(scorep, bvec, h, edges,
             wr2, wo2, b2r, wpr2, wpo2, bp2s,
             wr3, wo3, b3r, wpr3, wpo3, bp3s,
             wmt, bmr, out):
    s = scorep[0:1, :] + scorep[1:2, :] + bvec[...]
    iota = lax.broadcasted_iota(jnp.int32, (1, NP), 1)
    s = jnp.where(iota < N, s, jnp.float32(-3.0e38))
    m = jnp.max(s)
    idx = jnp.min(jnp.where(s == m, iota, NP))
    xn = h[pl.ds(idx, 1), :] * jnp.tanh(m)
    e0 = edges[0]
    e1 = edges[1]
    cf = jnp.sum(jnp.where((e0 == idx) & (e1 == idx), 1.0, 0.0))

    def gconv(v, wr, wo, br):
        y = cf * jnp.dot(v, wr[...], preferred_element_type=jnp.float32)
        y += jnp.dot(v, wo[...], preferred_element_type=jnp.float32)
        return jnp.maximum(y + br[...], 0.0)

    g2 = gconv(xn, wr2, wo2, b2r)
    s2 = cf * jnp.sum(g2 * wpr2[...]) + jnp.sum(g2 * wpo2[...]) + bp2s[0, 0]
    xn2 = g2 * jnp.tanh(s2)
    g3 = gconv(xn2, wr3, wo3, b3r)
    s3 = cf * jnp.sum(g3 * wpr3[...]) + jnp.sum(g3 * wpo3[...]) + bp3s[0, 0]
    xn3 = g3 * jnp.tanh(s3)
    t = xn + xn2 + xn3
    o0 = jnp.sum(t * wmt[0:1, :]) + bmr[0, 0]
    o1 = jnp.sum(t * wmt[1:2, :]) + bmr[0, 1]
    out[...] = jnp.concatenate([o0.reshape(1, 1), o1.reshape(1, 1)], axis=1)


def _k4(scorep, bvec, h, edges, *ws):
    return pl.pallas_call(
        _k4_body,
        out_shape=jax.ShapeDtypeStruct((1, 2), jnp.float32),
    )(scorep, bvec, h, edges, *ws)


def kernel(x, edge_index, batch, Wr1, Wo1, b1, Wpr1, Wpo1, bp1,
           Wr2, Wo2, b2, Wpr2, Wpo2, bp2, Wr3, Wo3, b3, Wpr3, Wpo3, bp3,
           Wm, bm):
    src = edge_index[0]
    dst = edge_index[1]
    pad = jnp.full((EP - E,), N, jnp.int32)
    srcp = jnp.concatenate([src.astype(jnp.int32), pad]).reshape(NW, RPW, 128)
    dstp = jnp.concatenate([dst.astype(jnp.int32), pad]).reshape(NW, RPW, 128)
    xpad = jnp.concatenate([x, jnp.zeros((NP - N, F), jnp.float32)], axis=0)
    zeros2d = jnp.zeros((NP, F), jnp.float32)

    aggp = _k1(xpad, srcp, dstp, zeros2d)

    h, a3, b3v = _k2(aggp, xpad,
                     Wr1, Wo1, b1.reshape(1, H),
                     Wpr1.reshape(1, H), Wpo1.reshape(1, H),
                     bp1.reshape(1, 1))
    a1 = a3.reshape(NP)
    bvec = b3v.reshape(1, NP)

    scorep = _k3(a1, srcp, dstp)

    edges = edge_index.astype(jnp.int32).reshape(2, E // 128, 128)
    wmt = (Wm[:H] + Wm[H:]).T
    return _k4(scorep, bvec, h, edges,
               Wr2, Wo2, b2.reshape(1, H), Wpr2.reshape(1, H),
               Wpo2.reshape(1, H), bp2.reshape(1, 1),
               Wr3, Wo3, b3.reshape(1, H), Wpr3.reshape(1, H),
               Wpo3.reshape(1, H), bp3.reshape(1, 1),
               wmt, bm.reshape(1, 2))
